# baseline (speedup 1.0000x reference)
import functools

import jax
import jax.numpy as jnp
from jax import lax
from jax.experimental import pallas as pl
from jax.experimental.pallas import tpu as pltpu
from jax.experimental.pallas import tpu_sc as plsc

NUM_CLASSES = 1000000
EMBED_DIM = 32
BATCH = 16384
FIELDS = 26

TOTAL = BATCH * FIELDS
NUM_CORES = 2
NUM_SUBCORES = 16
NW = NUM_CORES * NUM_SUBCORES

_MESH = plsc.VectorSubcoreMesh(core_axis_name="c", subcore_axis_name="s")

TPW = 244

@functools.partial(
    pl.kernel,
    mesh=_MESH,
    out_type=jax.ShapeDtypeStruct((NUM_CLASSES * EMBED_DIM,), jnp.float32),
    scratch_types=[
        pltpu.VMEM((4, 32, 128), jnp.float32),
        pltpu.VMEM((4096,), jnp.float32),
        pltpu.VMEM((4096,), jnp.float32),
        pltpu.VMEM((4096,), jnp.float32),
        pltpu.VMEM((4096,), jnp.float32),
        pltpu.VMEM((4, 8, 128), jnp.float32),
    ]
    + [pltpu.SemaphoreType.DMA] * 8,
    compiler_params=pltpu.CompilerParams(
        use_tc_tiling_on_sc=True, needs_layout_passes=False),
)
def _convert(tmat, trm, inb, outb0, outb1, outb2, outb3, tailb,
             isem0, isem1, isem2, isem3, osem0, osem1, osem2, osem3):
    wid = lax.axis_index("s") * NUM_CORES + lax.axis_index("c")
    base_c = wid * (TPW * 128)
    isems = (isem0, isem1, isem2, isem3)
    osems = (osem0, osem1, osem2, osem3)
    outbs = (outb0, outb1, outb2, outb3)
    idx_base = lax.iota(jnp.int32, 16) * 32
    NB = 4

    def in_copy(step, b):
        c0 = base_c + step * 128
        return pltpu.make_async_copy(
            tmat.at[pl.ds(0, 32), pl.ds(c0, 128)], inb.at[b], isems[b])

    def out_copy(step, b):
        c0 = base_c + step * 128
        return pltpu.make_async_copy(
            outbs[b], trm.at[pl.ds(c0 * 32, 4096)], osems[b])

    def transpose(b):
        def vbody(v, carry):
            idxv = idx_base + 512 * v
            vals = [None] * 32
            for q in range(32):
                vals[q] = inb[b, q, pl.ds(16 * v, 16)]
                if q >= 8:
                    plsc.store_scatter(outbs[b], [idxv + (q - 8)], vals[q - 8])
            for q in range(24, 32):
                plsc.store_scatter(outbs[b], [idxv + q], vals[q])
            return carry

        lax.fori_loop(0, 8, vbody, 0)

    for b in range(NB):
        in_copy(b, b).start()

    def body(i, carry):
        for b in range(NB):
            step = NB * i + b
            in_copy(step, b).wait()

            @pl.when(i > 0)
            def _():
                out_copy(step - NB, b).wait()

            transpose(b)
            out_copy(step, b).start()

            @pl.when(i < TPW // NB - 1)
            def _():
                in_copy(step + NB, b).start()
        return carry

    lax.fori_loop(0, TPW // NB, body, 0)
    for b in range(NB):
        out_copy(TPW - NB + b, b).wait()

    @pl.when(wid < 5)
    def _():
        c0 = (7808 + wid) * 128
        for tr in range(4):
            pltpu.sync_copy(
                tmat.at[pl.ds(8 * tr, 8), pl.ds(c0, 128)],
                tailb.at[tr])

    @pl.when(wid < 4)
    def _():
        c0 = (7808 + wid) * 128
        for tr in range(4):
            for s in range(8):
                off = tr * 8 + s
                for v in range(8):
                    val = tailb[tr, s, pl.ds(16 * v, 16)]
                    plsc.store_scatter(
                        outb0, [idx_base + (512 * v + off)], val)
        pltpu.sync_copy(outb0.at[pl.ds(0, 4096)],
                        trm.at[pl.ds(c0 * 32, 4096)])

    @pl.when(wid == 4)
    def _():
        c0 = (7808 + wid) * 128
        for tr in range(4):
            for s in range(8):
                off = tr * 8 + s
                for v in range(4):
                    val = tailb[tr, s, pl.ds(16 * v, 16)]
                    plsc.store_scatter(
                        outb0, [idx_base + (512 * v + off)], val)
        pltpu.sync_copy(outb0.at[pl.ds(0, 2048)],
                        trm.at[pl.ds(c0 * 32, 2048)])


PER_W = TOTAL // NW
CHUNK = 832
NCHUNK = PER_W // CHUNK
NBUF = 4


@functools.partial(
    pl.kernel,
    mesh=_MESH,
    out_type=jax.ShapeDtypeStruct((TOTAL, EMBED_DIM), jnp.float32),
    scratch_types=[
        pltpu.VMEM((PER_W,), jnp.int32),
        pltpu.VMEM((NBUF, CHUNK, EMBED_DIM), jnp.float32),
    ]
    + [pltpu.SemaphoreType.DMA] * (2 * NBUF),
    compiler_params=pltpu.CompilerParams(use_tc_tiling_on_sc=False),
)
def _emb_lookup(idx_hbm, table_hbm, out_hbm, idx_v, rows_v, *sems):
    gsems = sems[:NBUF]
    wsems = sems[NBUF:]
    wid = lax.axis_index("s") * NUM_CORES + lax.axis_index("c")
    base = wid * PER_W

    pltpu.sync_copy(idx_hbm.at[pl.ds(base, PER_W)], idx_v)

    def gather(i):
        b = i % NBUF
        return pltpu.async_copy(
            table_hbm.at[idx_v.at[pl.ds(i * CHUNK, CHUNK)]],
            rows_v.at[b], gsems[b])

    def put(i):
        b = i % NBUF
        return pltpu.async_copy(
            rows_v.at[b], out_hbm.at[pl.ds(base + i * CHUNK, CHUNK)],
            wsems[b])

    ghandles = [None] * NCHUNK
    whandles = [None] * NCHUNK
    for i in range(NBUF):
        ghandles[i] = gather(i)
    for i in range(NCHUNK):
        ghandles[i].wait()
        whandles[i] = put(i)
        if i + NBUF < NCHUNK:
            whandles[i].wait()
            ghandles[i + NBUF] = gather(i + NBUF)
    for i in range(NCHUNK - NBUF, NCHUNK):
        whandles[i].wait()


BLOCKS = FIELDS * (BATCH // 512)
BPW = BLOCKS // NW
NBUF_C = 2


@functools.partial(
    pl.kernel,
    mesh=_MESH,
    out_type=jax.ShapeDtypeStruct((FIELDS, EMBED_DIM, BATCH), jnp.float32),
    scratch_types=[pltpu.VMEM((16384,), jnp.float32)] * NBUF_C
    + [pltpu.VMEM((EMBED_DIM, 512), jnp.float32)] * NBUF_C
    + [pltpu.SemaphoreType.DMA] * (2 * NBUF_C),
    compiler_params=pltpu.CompilerParams(
        use_tc_tiling_on_sc=True, needs_layout_passes=False),
)
def _retile(y1d, out_hbm, *refs):
    inbs = refs[:NBUF_C]
    outbs = refs[NBUF_C:2 * NBUF_C]
    isems = refs[2 * NBUF_C:3 * NBUF_C]
    osems = refs[3 * NBUF_C:]
    wid = lax.axis_index("s") * NUM_CORES + lax.axis_index("c")
    base_blk = wid * BPW
    d_idx = [lax.iota(jnp.int32, 16) + 16 * h for h in range(2)]
    czero = jnp.full((16,), 0, jnp.int32)

    def in_copy(k, b):
        blk = base_blk + k
        return pltpu.make_async_copy(
            y1d.at[pl.ds(blk * 16384, 16384)], inbs[b], isems[b])

    def out_copies(k, b):
        blk = base_blk + k
        f = blk // 32
        tc0 = (blk % 32) * 512
        return [
            pltpu.make_async_copy(
                outbs[b].at[pl.ds(8 * tr, 8), pl.ds(0, 512)],
                out_hbm.at[f, pl.ds(8 * tr, 8), pl.ds(tc0, 512)],
                osems[b])
            for tr in range(4)
        ]

    def transpose(b):
        def cbody(co, carry):
            vals = [None] * 32
            cvs = [None] * 16
            for j in range(16):
                c = co * 16 + j
                cvs[j] = czero + c
                for h in range(2):
                    vals[2 * j + h] = inbs[b][pl.ds(c * EMBED_DIM + 16 * h, 16)]
                if j >= 4:
                    for h in range(2):
                        plsc.store_scatter(
                            outbs[b], [d_idx[h], cvs[j - 4]],
                            vals[2 * (j - 4) + h])
            for j in range(12, 16):
                for h in range(2):
                    plsc.store_scatter(
                        outbs[b], [d_idx[h], cvs[j]], vals[2 * j + h])
            return carry

        lax.fori_loop(0, 32, cbody, 0)

    for b in range(NBUF_C):
        in_copy(b, b).start()

    def body(i, carry):
        for b in range(NBUF_C):
            k = NBUF_C * i + b
            in_copy(k, b).wait()

            @pl.when(i > 0)
            def _():
                for c in out_copies(k - NBUF_C, b):
                    c.wait()

            transpose(b)
            for c in out_copies(k, b):
                c.start()

            @pl.when(i < BPW // NBUF_C - 1)
            def _():
                in_copy(k + NBUF_C, b).start()
        return carry

    lax.fori_loop(0, BPW // NBUF_C, body, 0)
    for b in range(NBUF_C):
        for c in out_copies(BPW - NBUF_C + b, b):
            c.wait()


def kernel(x, embed_map):
    tmat = embed_map.T
    trm = _convert(tmat)
    table = trm.reshape(NUM_CLASSES, EMBED_DIM)
    flat = jnp.transpose(x).reshape(TOTAL)
    y = _emb_lookup(flat, table)
    out3 = _retile(y.reshape(TOTAL * EMBED_DIM))
    return jnp.transpose(out3, (2, 0, 1))


# speedup vs baseline: 1.0016x; 1.0016x over previous
import functools

import jax
import jax.numpy as jnp
from jax import lax
from jax.experimental import pallas as pl
from jax.experimental.pallas import tpu as pltpu
from jax.experimental.pallas import tpu_sc as plsc

NUM_CLASSES = 1000000
EMBED_DIM = 32
BATCH = 16384
FIELDS = 26

TOTAL = BATCH * FIELDS
NUM_CORES = 2
NUM_SUBCORES = 16
NW = NUM_CORES * NUM_SUBCORES

_MESH = plsc.VectorSubcoreMesh(core_axis_name="c", subcore_axis_name="s")

TPW = 244

@functools.partial(
    pl.kernel,
    mesh=_MESH,
    out_type=jax.ShapeDtypeStruct((NUM_CLASSES * EMBED_DIM,), jnp.float32),
    scratch_types=[
        pltpu.VMEM((4, 32, 128), jnp.float32),
        pltpu.VMEM((4096,), jnp.float32),
        pltpu.VMEM((4096,), jnp.float32),
        pltpu.VMEM((4096,), jnp.float32),
        pltpu.VMEM((4096,), jnp.float32),
        pltpu.VMEM((4, 8, 128), jnp.float32),
    ]
    + [pltpu.SemaphoreType.DMA] * 8,
    compiler_params=pltpu.CompilerParams(
        use_tc_tiling_on_sc=True, needs_layout_passes=False),
)
def _convert(tmat, trm, inb, outb0, outb1, outb2, outb3, tailb,
             isem0, isem1, isem2, isem3, osem0, osem1, osem2, osem3):
    wid = lax.axis_index("s") * NUM_CORES + lax.axis_index("c")
    base_c = wid * (TPW * 128)
    isems = (isem0, isem1, isem2, isem3)
    osems = (osem0, osem1, osem2, osem3)
    outbs = (outb0, outb1, outb2, outb3)
    idx_base = lax.iota(jnp.int32, 16) * 32
    NB = 4

    def in_copy(step, b):
        c0 = base_c + step * 128
        return pltpu.make_async_copy(
            tmat.at[pl.ds(0, 32), pl.ds(c0, 128)], inb.at[b], isems[b])

    def out_copy(step, b):
        c0 = base_c + step * 128
        return pltpu.make_async_copy(
            outbs[b], trm.at[pl.ds(c0 * 32, 4096)], osems[b])

    def transpose(b):
        def vbody(v, carry):
            idxv = idx_base + 512 * v
            vals = [None] * 32
            for q in range(32):
                vals[q] = inb[b, q, pl.ds(16 * v, 16)]
                if q >= 8:
                    plsc.store_scatter(outbs[b], [idxv + (q - 8)], vals[q - 8])
            for q in range(24, 32):
                plsc.store_scatter(outbs[b], [idxv + q], vals[q])
            return carry

        lax.fori_loop(0, 8, vbody, 0)

    for b in range(NB):
        in_copy(b, b).start()

    def body(i, carry):
        for b in range(NB):
            step = NB * i + b
            in_copy(step, b).wait()

            @pl.when(i > 0)
            def _():
                out_copy(step - NB, b).wait()

            transpose(b)
            out_copy(step, b).start()

            @pl.when(i < TPW // NB - 1)
            def _():
                in_copy(step + NB, b).start()
        return carry

    lax.fori_loop(0, TPW // NB, body, 0)
    for b in range(NB):
        out_copy(TPW - NB + b, b).wait()

    @pl.when(wid < 5)
    def _():
        c0 = (7808 + wid) * 128
        for tr in range(4):
            pltpu.sync_copy(
                tmat.at[pl.ds(8 * tr, 8), pl.ds(c0, 128)],
                tailb.at[tr])

    @pl.when(wid < 4)
    def _():
        c0 = (7808 + wid) * 128
        for tr in range(4):
            for s in range(8):
                off = tr * 8 + s
                for v in range(8):
                    val = tailb[tr, s, pl.ds(16 * v, 16)]
                    plsc.store_scatter(
                        outb0, [idx_base + (512 * v + off)], val)
        pltpu.sync_copy(outb0.at[pl.ds(0, 4096)],
                        trm.at[pl.ds(c0 * 32, 4096)])

    @pl.when(wid == 4)
    def _():
        c0 = (7808 + wid) * 128
        for tr in range(4):
            for s in range(8):
                off = tr * 8 + s
                for v in range(4):
                    val = tailb[tr, s, pl.ds(16 * v, 16)]
                    plsc.store_scatter(
                        outb0, [idx_base + (512 * v + off)], val)
        pltpu.sync_copy(outb0.at[pl.ds(0, 2048)],
                        trm.at[pl.ds(c0 * 32, 2048)])


PER_W = TOTAL // NW
CHUNK = 832
NCHUNK = PER_W // CHUNK
NBUF = 4


@functools.partial(
    pl.kernel,
    mesh=_MESH,
    out_type=jax.ShapeDtypeStruct((TOTAL, EMBED_DIM), jnp.float32),
    scratch_types=[
        pltpu.VMEM((PER_W,), jnp.int32),
        pltpu.VMEM((NBUF, CHUNK, EMBED_DIM), jnp.float32),
    ]
    + [pltpu.SemaphoreType.DMA] * (2 * NBUF),
    compiler_params=pltpu.CompilerParams(use_tc_tiling_on_sc=False),
)
def _emb_lookup(idx_hbm, table_hbm, out_hbm, idx_v, rows_v, *sems):
    gsems = sems[:NBUF]
    wsems = sems[NBUF:]
    wid = lax.axis_index("s") * NUM_CORES + lax.axis_index("c")
    base = wid * PER_W

    pltpu.sync_copy(idx_hbm.at[pl.ds(base, PER_W)], idx_v)

    def gather(i):
        b = i % NBUF
        return pltpu.async_copy(
            table_hbm.at[idx_v.at[pl.ds(i * CHUNK, CHUNK)]],
            rows_v.at[b], gsems[b])

    def put(i):
        b = i % NBUF
        return pltpu.async_copy(
            rows_v.at[b], out_hbm.at[pl.ds(base + i * CHUNK, CHUNK)],
            wsems[b])

    ghandles = [None] * NCHUNK
    whandles = [None] * NCHUNK
    for i in range(NBUF):
        ghandles[i] = gather(i)
    for i in range(NCHUNK):
        ghandles[i].wait()
        whandles[i] = put(i)
        if i + NBUF < NCHUNK:
            whandles[i].wait()
            ghandles[i + NBUF] = gather(i + NBUF)
    for i in range(NCHUNK - NBUF, NCHUNK):
        whandles[i].wait()


BLOCKS = FIELDS * (BATCH // 128)
BPW = BLOCKS // NW
NBUF_C = 4


@functools.partial(
    pl.kernel,
    mesh=_MESH,
    out_type=jax.ShapeDtypeStruct((FIELDS, EMBED_DIM, BATCH), jnp.float32),
    scratch_types=[pltpu.VMEM((4096,), jnp.float32)] * NBUF_C
    + [pltpu.VMEM((EMBED_DIM, 128), jnp.float32)] * NBUF_C
    + [pltpu.SemaphoreType.DMA] * (2 * NBUF_C),
    compiler_params=pltpu.CompilerParams(
        use_tc_tiling_on_sc=True, needs_layout_passes=False),
)
def _retile(y1d, out_hbm, *refs):
    inbs = refs[:NBUF_C]
    outbs = refs[NBUF_C:2 * NBUF_C]
    isems = refs[2 * NBUF_C:3 * NBUF_C]
    osems = refs[3 * NBUF_C:]
    wid = lax.axis_index("s") * NUM_CORES + lax.axis_index("c")
    base_blk = wid * BPW
    d_idx = [lax.iota(jnp.int32, 16) + 16 * h for h in range(2)]
    czero = jnp.full((16,), 0, jnp.int32)

    def in_copy(k, b):
        blk = base_blk + k
        return pltpu.make_async_copy(
            y1d.at[pl.ds(blk * 4096, 4096)], inbs[b], isems[b])

    def out_copy(k, b):
        blk = base_blk + k
        f = blk // 128
        tc = blk % 128
        return pltpu.make_async_copy(
            outbs[b],
            out_hbm.at[f, pl.ds(0, EMBED_DIM), pl.ds(tc * 128, 128)],
            osems[b])

    def transpose(b):
        def cbody(co, carry):
            vals = [None] * 32
            cvs = [None] * 16
            for j in range(16):
                c = co * 16 + j
                cvs[j] = czero + c
                for h in range(2):
                    vals[2 * j + h] = inbs[b][pl.ds(c * EMBED_DIM + 16 * h, 16)]
                if j >= 4:
                    for h in range(2):
                        plsc.store_scatter(
                            outbs[b], [d_idx[h], cvs[j - 4]],
                            vals[2 * (j - 4) + h])
            for j in range(12, 16):
                for h in range(2):
                    plsc.store_scatter(
                        outbs[b], [d_idx[h], cvs[j]], vals[2 * j + h])
            return carry

        lax.fori_loop(0, 8, cbody, 0)

    for b in range(NBUF_C):
        in_copy(b, b).start()

    def body(i, carry):
        for b in range(NBUF_C):
            k = NBUF_C * i + b
            in_copy(k, b).wait()

            @pl.when(i > 0)
            def _():
                out_copy(k - NBUF_C, b).wait()

            transpose(b)
            out_copy(k, b).start()

            @pl.when(i < BPW // NBUF_C - 1)
            def _():
                in_copy(k + NBUF_C, b).start()
        return carry

    lax.fori_loop(0, BPW // NBUF_C, body, 0)
    for b in range(NBUF_C):
        out_copy(BPW - NBUF_C + b, b).wait()


def kernel(x, embed_map):
    tmat = embed_map.T
    trm = _convert(tmat)
    table = trm.reshape(NUM_CLASSES, EMBED_DIM)
    flat = jnp.transpose(x).reshape(TOTAL)
    y = _emb_lookup(flat, table)
    out3 = _retile(y.reshape(TOTAL * EMBED_DIM))
    return jnp.transpose(out3, (2, 0, 1))
